# baseline (speedup 1.0000x reference)
import functools

import jax
import jax.numpy as jnp
from jax import lax
from jax.experimental import pallas as pl
from jax.experimental.pallas import tpu as pltpu
from jax.experimental.pallas import tpu_sc as plsc

_NC = 2
_NS = 16
_NW = _NC * _NS
_L = 16

_GC = 128


@functools.partial(jax.jit, static_argnames=())
def _run(data, R, S):
  B = data.shape[0]
  D = R.shape[1]
  bpw = B // _NW

  mesh = plsc.VectorSubcoreMesh(
      core_axis_name="c", subcore_axis_name="s",
      num_cores=_NC, num_subcores=_NS)

  @functools.partial(
      pl.kernel,
      out_type=jax.ShapeDtypeStruct((B,), jnp.float32),
      mesh=mesh,
      compiler_params=pltpu.CompilerParams(
          needs_layout_passes=False, use_tc_tiling_on_sc=False),
      scratch_types=[
          pltpu.VMEM((bpw * 2,), jnp.int32),
          pltpu.VMEM((bpw,), jnp.int32),
          pltpu.VMEM((bpw,), jnp.int32),
          pltpu.VMEM((bpw, D), jnp.float32),
          pltpu.VMEM((bpw, D), jnp.float32),
          pltpu.VMEM((bpw,), jnp.float32),
          pltpu.SemaphoreType.DMA,
      ],
  )
  def sc_kernel(data_hbm, r_hbm, s_hbm, out_hbm,
                dv, tv, uv, rv, sv, ov, sem):
    wid = lax.axis_index("s") * _NC + lax.axis_index("c")
    base = wid * bpw
    lane = lax.iota(jnp.int32, _L)

    pltpu.sync_copy(data_hbm.at[pl.ds(base * 2, bpw * 2)], dv)

    def deinterleave(b, carry):
      flat = (lane + b * _L) * 2
      off = pl.multiple_of(b * _L, _L)
      tv[pl.ds(off, _L)] = plsc.load_gather(dv, [flat])
      uv[pl.ds(off, _L)] = plsc.load_gather(dv, [flat + 1])
      return carry

    lax.fori_loop(0, bpw // _L, deinterleave, 0)

    copies = []
    for c in range(bpw // _GC):
      idx_t = tv.at[pl.ds(c * _GC, _GC)]
      idx_u = uv.at[pl.ds(c * _GC, _GC)]
      copies.append(pltpu.async_copy(
          r_hbm.at[idx_t], rv.at[pl.ds(c * _GC, _GC), :], sem))
      copies.append(pltpu.async_copy(
          s_hbm.at[idx_u], sv.at[pl.ds(c * _GC, _GC), :], sem))
    for cp in copies:
      cp.wait()

    def block(b, carry):
      row = lane + b * _L
      acc = jnp.zeros((_L,), jnp.float32)
      for k in range(D):
        col = jnp.full((_L,), k, jnp.int32)
        acc = acc + (plsc.load_gather(rv, [row, col]) *
                     plsc.load_gather(sv, [row, col]))
      off = pl.multiple_of(b * _L, _L)
      ov[pl.ds(off, _L)] = acc
      return carry

    lax.fori_loop(0, bpw // _L, block, 0)

    pltpu.sync_copy(ov, out_hbm.at[pl.ds(base, bpw)])

  return sc_kernel(data.reshape(-1), R, S)


def kernel(data, R, S):
  return _run(data, R, S)
